# baseline (device time: 71024 ns/iter reference)
import jax
import jax.numpy as jnp
from jax import lax
from jax.experimental import pallas as pl
from jax.experimental.pallas import tpu as pltpu

N_DEV = 32
M = 1024
N = 1024
NS = 4
COL = N // NS

RS_MASKS = (1, 8, 2, 4, 16)
RS_HALF = (512, 256, 128, 64, 32)
RS_OFF = (0, 512, 768, 896, 960)
AG_MASKS = tuple(reversed(RS_MASKS))
AG_SZ = (32, 64, 128, 256, 512)


def kernel(x, W1, W2):
    def body(x_ref, w1_ref, w2_ref, out_ref, acc, stage,
             send_sems, rs_sems, ag_sems):
        my = lax.axis_index("i")

        barrier = pltpu.get_barrier_semaphore()
        for m in RS_MASKS:
            pl.semaphore_signal(
                barrier, inc=1,
                device_id=(my ^ m,), device_id_type=pl.DeviceIdType.MESH,
            )

        bits = [jnp.where((my & m) != 0, 1, 0).astype(jnp.int32) for m in RS_MASKS]
        rs_lo = [jnp.int32(0)]
        rs_send_lo = []
        for r, half in enumerate(RS_HALF):
            rs_send_lo.append(rs_lo[r] + (1 - bits[r]) * half)
            rs_lo.append(rs_lo[r] + bits[r] * half)
        ag_lo = [rs_lo[-1]]
        for r, m in enumerate(AG_MASKS):
            b = bits[RS_MASKS.index(m)]
            ag_lo.append(ag_lo[r] - b * AG_SZ[r])

        def rs_desc(s, r):
            half = RS_HALF[r]
            cols = pl.ds(s * COL, COL)
            return pltpu.make_async_remote_copy(
                src_ref=acc.at[pl.ds(rs_send_lo[r], half), cols],
                dst_ref=stage.at[pl.ds(RS_OFF[r], half), cols],
                send_sem=send_sems.at[s],
                recv_sem=rs_sems.at[s, r],
                device_id=(my ^ RS_MASKS[r],),
                device_id_type=pl.DeviceIdType.MESH,
            )

        def ag_desc(s, r):
            seg = acc.at[pl.ds(ag_lo[r], AG_SZ[r]), pl.ds(s * COL, COL)]
            return pltpu.make_async_remote_copy(
                src_ref=seg, dst_ref=seg,
                send_sem=send_sems.at[s],
                recv_sem=ag_sems.at[s, r],
                device_id=(my ^ AG_MASKS[r],),
                device_id_type=pl.DeviceIdType.MESH,
            )

        xb = x_ref[...].astype(jnp.bfloat16)
        w1b = w1_ref[...].astype(jnp.bfloat16)
        h = jnp.dot(xb, w1b, preferred_element_type=jnp.float32)
        h = jnp.maximum(h, 0.0).astype(jnp.bfloat16)
        w2b = w2_ref[...].astype(jnp.bfloat16)
        inflight = {}
        for s in range(NS):
            p = jnp.dot(h, w2b[:, s * COL:(s + 1) * COL],
                        preferred_element_type=jnp.float32)
            acc[:, s * COL:(s + 1) * COL] = p.astype(jnp.bfloat16)
            if s == 0:
                pl.semaphore_wait(barrier, len(RS_MASKS))
            inflight[(s, 0)] = rs_desc(s, 0)
            inflight[(s, 0)].start()

        for r in range(5):
            half = RS_HALF[r]
            krows = pl.ds(rs_lo[r + 1], half)
            srows = pl.ds(RS_OFF[r], half)
            for s in range(NS):
                cols = pl.ds(s * COL, COL)
                inflight[(s, r)].wait_recv()
                acc[krows, cols] = acc[krows, cols] + stage[srows, cols]
                if r < 4:
                    inflight[(s, r)].wait_send()
                    inflight[(s, r + 1)] = rs_desc(s, r + 1)
                    inflight[(s, r + 1)].start()

        ag = {}
        for s in range(NS):
            inflight[(s, 4)].wait_send()
            ag[(s, 0)] = ag_desc(s, 0)
            ag[(s, 0)].start()
        for r in range(5):
            for s in range(NS):
                ag[(s, r)].wait_recv()
                if r < 4:
                    ag[(s, r)].wait_send()
                    ag[(s, r + 1)] = ag_desc(s, r + 1)
                    ag[(s, r + 1)].start()

        out_ref[...] = acc[...].astype(jnp.float32)
        for s in range(NS):
            ag[(s, 4)].wait_send()

    return pl.pallas_call(
        body,
        out_shape=jax.ShapeDtypeStruct((M, N), jnp.float32),
        in_specs=[
            pl.BlockSpec(memory_space=pltpu.VMEM),
            pl.BlockSpec(memory_space=pltpu.VMEM),
            pl.BlockSpec(memory_space=pltpu.VMEM),
        ],
        out_specs=pl.BlockSpec(memory_space=pltpu.VMEM),
        scratch_shapes=[
            pltpu.VMEM((M, N), jnp.bfloat16),
            pltpu.VMEM((M, N), jnp.bfloat16),
            pltpu.SemaphoreType.DMA((NS,)),
            pltpu.SemaphoreType.DMA((NS, 5)),
            pltpu.SemaphoreType.DMA((NS, 5)),
        ],
        compiler_params=pltpu.CompilerParams(collective_id=0),
    )(x, W1, W2)


# device time: 63622 ns/iter; 1.1163x vs baseline; 1.1163x over previous
import jax
import jax.numpy as jnp
from jax import lax
from jax.experimental import pallas as pl
from jax.experimental.pallas import tpu as pltpu

N_DEV = 32
M = 1024
N = 1024
NS = 2
COL = N // NS

RS_PHASES = (
    (4, 1, 256, (0, 256, 512)),
    (4, 8, 64, (768, 832, 896)),
    (2, 4, 32, (960,)),
)


def kernel(x, W1, W2):
    def body(x_ref, w1_ref, w2_ref, out_ref, acc, stage,
             send_sems, rs_sems, ag_sems):
        my = lax.axis_index("i")

        gs = [my & 3, (my >> 3) & 3, (my >> 2) & 1]
        lo = [jnp.int32(0)]
        for (radix, m, qt, _), g in zip(RS_PHASES, gs):
            lo.append(lo[-1] + g * qt)

        barrier = pltpu.get_barrier_semaphore()
        partners = [1, 2, 3, 8, 16, 24, 4]
        for m in partners:
            pl.semaphore_signal(
                barrier, inc=1,
                device_id=(my ^ m,), device_id_type=pl.DeviceIdType.MESH,
            )

        def rs_descs(s, p):
            radix, m, qt, offs = RS_PHASES[p]
            g = gs[p]
            cols = pl.ds(s * COL, COL)
            out = []
            for d in range(1, radix):
                j = g ^ d
                out.append(pltpu.make_async_remote_copy(
                    src_ref=acc.at[pl.ds(lo[p] + j * qt, qt), cols],
                    dst_ref=stage.at[pl.ds(offs[d - 1], qt), cols],
                    send_sem=send_sems.at[s, d - 1],
                    recv_sem=rs_sems.at[s, p, d - 1],
                    device_id=(my ^ (d * m),),
                    device_id_type=pl.DeviceIdType.MESH,
                ))
            return out

        ag_sz = {2: 32, 1: 64, 0: 256}
        ag_seg = {2: lo[3], 1: lo[3] - gs[2] * 32, 0: lo[1]}

        def ag_descs(s, p):
            radix, m, _, _ = RS_PHASES[p]
            seg = acc.at[pl.ds(ag_seg[p], ag_sz[p]), pl.ds(s * COL, COL)]
            out = []
            for d in range(1, radix):
                out.append(pltpu.make_async_remote_copy(
                    src_ref=seg, dst_ref=seg,
                    send_sem=send_sems.at[s, d - 1],
                    recv_sem=ag_sems.at[s, p, d - 1],
                    device_id=(my ^ (d * m),),
                    device_id_type=pl.DeviceIdType.MESH,
                ))
            return out

        xb = x_ref[...].astype(jnp.bfloat16)
        w1b = w1_ref[...].astype(jnp.bfloat16)
        h = jnp.dot(xb, w1b, preferred_element_type=jnp.float32)
        h = jnp.maximum(h, 0.0).astype(jnp.bfloat16)
        w2b = w2_ref[...].astype(jnp.bfloat16)
        inflight = {}
        for s in range(NS):
            p = jnp.dot(h, w2b[:, s * COL:(s + 1) * COL],
                        preferred_element_type=jnp.float32)
            acc[:, s * COL:(s + 1) * COL] = p.astype(jnp.bfloat16)
            if s == 0:
                pl.semaphore_wait(barrier, len(partners))
            inflight[s] = rs_descs(s, 0)
            for d_ in inflight[s]:
                d_.start()

        for p, (radix, m, qt, offs) in enumerate(RS_PHASES):
            krows = pl.ds(lo[p] + gs[p] * qt, qt)
            for s in range(NS):
                cols = pl.ds(s * COL, COL)
                for d_ in inflight[s]:
                    d_.wait_recv()
                total = stage[pl.ds(offs[0], qt), cols]
                for off in offs[1:]:
                    total = total + stage[pl.ds(off, qt), cols]
                acc[krows, cols] = acc[krows, cols] + total
                for d_ in inflight[s]:
                    d_.wait_send()
                if p < 2:
                    inflight[s] = rs_descs(s, p + 1)
                    for d_ in inflight[s]:
                        d_.start()

        for s in range(NS):
            inflight[s] = ag_descs(s, 2)
            for d_ in inflight[s]:
                d_.start()
        for p in (2, 1, 0):
            for s in range(NS):
                for d_ in inflight[s]:
                    d_.wait_recv()
                for d_ in inflight[s]:
                    d_.wait_send()
                if p > 0:
                    inflight[s] = ag_descs(s, p - 1)
                    for d_ in inflight[s]:
                        d_.start()

        out_ref[...] = acc[...].astype(jnp.float32)

    return pl.pallas_call(
        body,
        out_shape=jax.ShapeDtypeStruct((M, N), jnp.float32),
        in_specs=[
            pl.BlockSpec(memory_space=pltpu.VMEM),
            pl.BlockSpec(memory_space=pltpu.VMEM),
            pl.BlockSpec(memory_space=pltpu.VMEM),
        ],
        out_specs=pl.BlockSpec(memory_space=pltpu.VMEM),
        scratch_shapes=[
            pltpu.VMEM((M, N), jnp.bfloat16),
            pltpu.VMEM((M, N), jnp.bfloat16),
            pltpu.SemaphoreType.DMA((NS, 3)),
            pltpu.SemaphoreType.DMA((NS, 3, 3)),
            pltpu.SemaphoreType.DMA((NS, 3, 3)),
        ],
        compiler_params=pltpu.CompilerParams(collective_id=0),
    )(x, W1, W2)


# device time: 61312 ns/iter; 1.1584x vs baseline; 1.0377x over previous
import jax
import jax.numpy as jnp
from jax import lax
from jax.experimental import pallas as pl
from jax.experimental.pallas import tpu as pltpu

N_DEV = 32
M = 1024
N = 1024
NS = 2
COL = N // NS

RS_PHASES = (
    ((4, 1, 256, (0, 256, 512)), (4, 8, 64, (768, 832, 896)), (2, 4, 32, (960,))),
    ((4, 8, 256, (0, 256, 512)), (4, 1, 64, (768, 832, 896)), (2, 4, 32, (960,))),
)
BARRIER_MASKS = (1, 2, 3, 8, 16, 24, 4)


def kernel(x, W1, W2):
    def body(x_ref, w1_ref, w2_ref, out_ref, acc, stage,
             send_sems, rs_sems, ag_sems):
        my = lax.axis_index("i")

        def g_of(mask):
            return {1: my & 3, 8: (my >> 3) & 3, 4: (my >> 2) & 1}[mask]

        gs = [[g_of(ph[1]) for ph in RS_PHASES[s]] for s in range(NS)]
        lo = []
        for s in range(NS):
            chain = [jnp.int32(0)]
            for (radix, m, qt, _), g in zip(RS_PHASES[s], gs[s]):
                chain.append(chain[-1] + g * qt)
            lo.append(chain)

        barrier = pltpu.get_barrier_semaphore()
        for m in BARRIER_MASKS:
            pl.semaphore_signal(
                barrier, inc=1,
                device_id=(my ^ m,), device_id_type=pl.DeviceIdType.MESH,
            )

        def rs_descs(s, p):
            radix, m, qt, offs = RS_PHASES[s][p]
            g = gs[s][p]
            cols = pl.ds(s * COL, COL)
            out = []
            for d in range(1, radix):
                j = g ^ d
                out.append(pltpu.make_async_remote_copy(
                    src_ref=acc.at[pl.ds(lo[s][p] + j * qt, qt), cols],
                    dst_ref=stage.at[pl.ds(offs[d - 1], qt), cols],
                    send_sem=send_sems.at[s, d - 1],
                    recv_sem=rs_sems.at[s, p, d - 1],
                    device_id=(my ^ (d * m),),
                    device_id_type=pl.DeviceIdType.MESH,
                ))
            return out

        def ag_seg(s, p):
            radix, m, qt, _ = RS_PHASES[s][p]
            starts = {2: lo[s][3], 1: lo[s][2], 0: lo[s][1]}
            sizes = {2: 32, 1: 64, 0: 256}
            return starts[p], sizes[p]

        def ag_descs(s, p):
            radix, m, qt, _ = RS_PHASES[s][p]
            start, sz = ag_seg(s, p)
            seg = acc.at[pl.ds(start, sz), pl.ds(s * COL, COL)]
            out = []
            for d in range(1, radix):
                out.append(pltpu.make_async_remote_copy(
                    src_ref=seg, dst_ref=seg,
                    send_sem=send_sems.at[s, d - 1],
                    recv_sem=ag_sems.at[s, p, d - 1],
                    device_id=(my ^ (d * m),),
                    device_id_type=pl.DeviceIdType.MESH,
                ))
            return out

        xb = x_ref[...].astype(jnp.bfloat16)
        w1b = w1_ref[...].astype(jnp.bfloat16)
        h = jnp.dot(xb, w1b, preferred_element_type=jnp.float32)
        h = jnp.maximum(h, 0.0).astype(jnp.bfloat16)
        w2b = w2_ref[...].astype(jnp.bfloat16)
        inflight = {}
        for s in range(NS):
            p = jnp.dot(h, w2b[:, s * COL:(s + 1) * COL],
                        preferred_element_type=jnp.float32)
            acc[:, s * COL:(s + 1) * COL] = p.astype(jnp.bfloat16)
            if s == 0:
                pl.semaphore_wait(barrier, len(BARRIER_MASKS))
            inflight[s] = rs_descs(s, 0)
            for d_ in inflight[s]:
                d_.start()

        for p in range(3):
            for s in range(NS):
                radix, m, qt, offs = RS_PHASES[s][p]
                krows = pl.ds(lo[s][p] + gs[s][p] * qt, qt)
                cols = pl.ds(s * COL, COL)
                for d_ in inflight[s]:
                    d_.wait_recv()
                total = stage[pl.ds(offs[0], qt), cols]
                for off in offs[1:]:
                    total = total + stage[pl.ds(off, qt), cols]
                acc[krows, cols] = acc[krows, cols] + total
                for d_ in inflight[s]:
                    d_.wait_send()
                if p < 2:
                    inflight[s] = rs_descs(s, p + 1)
                    for d_ in inflight[s]:
                        d_.start()

        for s in range(NS):
            inflight[s] = ag_descs(s, 2)
            for d_ in inflight[s]:
                d_.start()
        for p in (2, 1, 0):
            for s in range(NS):
                for d_ in inflight[s]:
                    d_.wait_recv()
                for d_ in inflight[s]:
                    d_.wait_send()
                if p > 0:
                    inflight[s] = ag_descs(s, p - 1)
                    for d_ in inflight[s]:
                        d_.start()

        out_ref[...] = acc[...].astype(jnp.float32)

    return pl.pallas_call(
        body,
        out_shape=jax.ShapeDtypeStruct((M, N), jnp.float32),
        in_specs=[
            pl.BlockSpec(memory_space=pltpu.VMEM),
            pl.BlockSpec(memory_space=pltpu.VMEM),
            pl.BlockSpec(memory_space=pltpu.VMEM),
        ],
        out_specs=pl.BlockSpec(memory_space=pltpu.VMEM),
        scratch_shapes=[
            pltpu.VMEM((M, N), jnp.bfloat16),
            pltpu.VMEM((M, N), jnp.bfloat16),
            pltpu.SemaphoreType.DMA((NS, 3)),
            pltpu.SemaphoreType.DMA((NS, 3, 3)),
            pltpu.SemaphoreType.DMA((NS, 3, 3)),
        ],
        compiler_params=pltpu.CompilerParams(collective_id=0),
    )(x, W1, W2)
